# Initial kernel scaffold; baseline (speedup 1.0000x reference)
import jax
import jax.numpy as jnp
from jax.experimental import pallas as pl


def kernel(seq1, seq2, table, gamma, beta, W, b):
    raise NotImplementedError("write your pallas kernel here")


# speedup vs baseline: 1.3652x; 1.3652x over previous
import jax
import jax.numpy as jnp
from jax import lax
from jax.experimental import pallas as pl
from jax.experimental.pallas import tpu as pltpu
from jax.experimental.pallas import tpu_sc as plsc

NC, NS, LANES = 2, 16, 16
NW = NC * NS

D = 64
NJ = D // LANES
CH = 104
NCHUNK = 2
LP = CH * NCHUNK


def _rsqrt16(x):
    i = plsc.bitcast(x, jnp.int32)
    i = jnp.full((LANES,), 0x5F3759DF, jnp.int32) - lax.shift_right_logical(i, 1)
    y = plsc.bitcast(i, jnp.float32)
    for _ in range(3):
        y = y * (1.5 - 0.5 * x * y * y)
    return y


def _sc_pool(seq, table, gamma, beta, rows_per_worker):
    b2 = seq.shape[0]

    def body(seq_hbm, table_hbm, gamma_hbm, beta_hbm, out_hbm,
             idx0_v, idx1_v, rows_v, outbuf_v, gam_v, bet_v, sem):
        idx_refs = (idx0_v, idx1_v)
        wid = lax.axis_index("s") * NC + lax.axis_index("c")
        base = wid * rows_per_worker
        pltpu.sync_copy(gamma_hbm, gam_v)
        pltpu.sync_copy(beta_hbm, bet_v)

        def row_body(rl, _):
            r = base + rl
            pltpu.sync_copy(seq_hbm.at[r, 0], idx0_v)
            pltpu.sync_copy(seq_hbm.at[r, 1], idx1_v)
            h0 = pltpu.async_copy(table_hbm.at[idx0_v], rows_v.at[0], sem)
            h1 = pltpu.async_copy(table_hbm.at[idx1_v], rows_v.at[1], sem)
            h0.wait()
            h1.wait()

            cnt = jnp.zeros((LANES,), jnp.float32)
            for c in range(NCHUNK):
                for g in range(CH // LANES):
                    t16 = idx_refs[c][pl.ds(g * LANES, LANES)]
                    cnt += jnp.where(t16 != 0, 1.0, 0.0)
                t16 = idx_refs[c][pl.ds(CH - LANES, LANES)]
                tail = (lax.iota(jnp.int32, LANES) >= 8) & (t16 != 0)
                cnt += jnp.where(tail, 1.0, 0.0)
            cnt_tot = jnp.broadcast_to(jnp.sum(cnt), (LANES,))

            acc = [jnp.zeros((LANES,), jnp.float32) for _ in range(NJ)]
            smu = jnp.zeros((LANES,), jnp.float32)
            for c in range(NCHUNK):
                def tok_body(o, carry, c=c):
                    a0, a1, a2, a3, sm = carry
                    x = [rows_v[c, o, pl.ds(j * LANES, LANES)]
                         for j in range(NJ)]
                    s = (x[0] + x[1]) + (x[2] + x[3])
                    q = ((x[0] * x[0] + x[1] * x[1])
                         + (x[2] * x[2] + x[3] * x[3]))
                    st = jnp.sum(s)
                    qt = jnp.sum(q)
                    mu = st * (1.0 / D)
                    var = qt * (1.0 / D) - mu * mu
                    kv = _rsqrt16(jnp.broadcast_to(var + 1e-5, (LANES,)))
                    sm = sm + kv * mu
                    return (a0 + x[0] * kv, a1 + x[1] * kv,
                            a2 + x[2] * kv, a3 + x[3] * kv, sm)
                *acc, smu = lax.fori_loop(
                    0, CH, tok_body, tuple(acc) + (smu,), unroll=4)
                acc = list(acc)

            rdenom = 1.0 / jnp.maximum(cnt_tot, 1e-9)
            for j in range(NJ):
                gj = gam_v[pl.ds(j * LANES, LANES)]
                bj = bet_v[pl.ds(j * LANES, LANES)]
                outbuf_v[rl, pl.ds(j * LANES, LANES)] = (
                    (gj * (acc[j] - smu) + bj * cnt_tot) * rdenom)
            return ()

        lax.fori_loop(0, rows_per_worker, row_body, (), unroll=False)
        pltpu.sync_copy(outbuf_v, out_hbm.at[pl.ds(base, rows_per_worker)])

    mesh = plsc.VectorSubcoreMesh(
        core_axis_name="c", subcore_axis_name="s",
        num_cores=NC, num_subcores=NS)
    return pl.kernel(
        body,
        out_type=jax.ShapeDtypeStruct((b2, D), jnp.float32),
        mesh=mesh,
        compiler_params=pltpu.CompilerParams(
            needs_layout_passes=False, use_tc_tiling_on_sc=False),
        scratch_types=[
            pltpu.VMEM((CH,), jnp.int32),
            pltpu.VMEM((CH,), jnp.int32),
            pltpu.VMEM((NCHUNK, CH, D), jnp.float32),
            pltpu.VMEM((rows_per_worker, D), jnp.float32),
            pltpu.VMEM((D,), jnp.float32),
            pltpu.VMEM((D,), jnp.float32),
            pltpu.SemaphoreType.DMA,
        ],
    )(seq, table, gamma, beta)


def _tc_head(p1, p2, w, b2d):
    bh = p1.shape[0]

    def body(p1_ref, p2_ref, w_ref, b_ref, sim_ref, v1_ref, v2_ref):
        ww = w_ref[...]
        bb = b_ref[...]
        dn = (((1,), (1,)), ((), ()))
        v1 = lax.dot_general(p1_ref[...], ww, dn,
                             preferred_element_type=jnp.float32) + bb
        v2 = lax.dot_general(p2_ref[...], ww, dn,
                             preferred_element_type=jnp.float32) + bb
        v1_ref[...] = v1
        v2_ref[...] = v2
        n1 = jnp.maximum(jnp.sqrt(jnp.sum(v1 * v1, -1, keepdims=True)), 1e-8)
        n2 = jnp.maximum(jnp.sqrt(jnp.sum(v2 * v2, -1, keepdims=True)), 1e-8)
        sim_ref[...] = jnp.sum(v1 * v2, -1, keepdims=True) / (n1 * n2)

    return pl.pallas_call(
        body,
        out_shape=[
            jax.ShapeDtypeStruct((bh, 1), jnp.float32),
            jax.ShapeDtypeStruct((bh, D), jnp.float32),
            jax.ShapeDtypeStruct((bh, D), jnp.float32),
        ],
    )(p1, p2, w, b2d)


def kernel(seq1, seq2, table, gamma, beta, W, b):
    bh, seq_len = seq1.shape
    seq = jnp.concatenate([seq1, seq2], axis=0).astype(jnp.int32)
    seq = jnp.pad(seq, ((0, 0), (0, LP - seq_len)))
    seq = seq.reshape(2 * bh, NCHUNK, CH)
    pooled = _sc_pool(seq, table, gamma, beta, (2 * bh) // NW)
    sim2d, v1, v2 = _tc_head(pooled[:bh], pooled[bh:], W,
                             b.reshape(1, D))
    return (sim2d.reshape(bh), v1, v2)
